# baseline (speedup 1.0000x reference)
import functools

import jax
import jax.numpy as jnp
from jax import lax
from jax.experimental import pallas as pl
from jax.experimental.pallas import tpu as pltpu
from jax.experimental.pallas import tpu_sc as plsc

B_TOTAL = 1024 * 200
D_EMBED = 64
NW = 32
B_PER_W = B_TOTAL // NW
CHUNK = 128
NCHUNK = B_PER_W // CHUNK


def _gather_kernel(idx_hbm, table_hbm, out_hbm, idx_v, rows_v, sem):
    wid = lax.axis_index("s") * 2 + lax.axis_index("c")
    base = wid * B_PER_W
    pltpu.sync_copy(idx_hbm.at[pl.ds(base, B_PER_W)], idx_v)

    def body(g, carry):
        off = g * CHUNK
        pltpu.async_copy(
            table_hbm.at[idx_v.at[pl.ds(off, CHUNK)]], rows_v, sem
        ).wait()
        pltpu.sync_copy(rows_v, out_hbm.at[pl.ds(base + off, CHUNK)])
        return carry

    lax.fori_loop(0, NCHUNK, body, 0)


@jax.jit
def kernel(x, emb):
    idx = x.reshape(-1).astype(jnp.int32)
    mesh = plsc.VectorSubcoreMesh(core_axis_name="c", subcore_axis_name="s")
    out = pl.kernel(
        _gather_kernel,
        out_type=jax.ShapeDtypeStruct((B_TOTAL, D_EMBED), jnp.float32),
        mesh=mesh,
        scratch_types=[
            pltpu.VMEM((B_PER_W,), jnp.int32),
            pltpu.VMEM((CHUNK, D_EMBED), jnp.float32),
            pltpu.SemaphoreType.DMA,
        ],
        compiler_params=pltpu.CompilerParams(use_tc_tiling_on_sc=False),
    )(idx, emb)
    return out.reshape(x.shape[0], x.shape[1], D_EMBED)


# speedup vs baseline: 1.0461x; 1.0461x over previous
import jax
import jax.numpy as jnp
from jax import lax
from jax.experimental import pallas as pl
from jax.experimental.pallas import tpu as pltpu
from jax.experimental.pallas import tpu_sc as plsc

B_TOTAL = 1024 * 200
D_EMBED = 64
NW = 32
B_PER_W = B_TOTAL // NW
CHUNK = 128
NCHUNK = B_PER_W // CHUNK
NBUF = 10
LOOK = 5
NROUND = NCHUNK // NBUF


def _gather_kernel(idx_hbm, table_hbm, out_hbm, idx_v, rows_v, gsem, osem):
    wid = lax.axis_index("s") * 2 + lax.axis_index("c")
    base = wid * B_PER_W
    pltpu.sync_copy(idx_hbm.at[pl.ds(base, B_PER_W)], idx_v)

    def gather(g, b):
        return pltpu.make_async_copy(
            table_hbm.at[idx_v.at[pl.ds(g * CHUNK, CHUNK)]],
            rows_v.at[b],
            gsem.at[b],
        )

    def out_copy(g, b):
        return pltpu.make_async_copy(
            rows_v.at[b],
            out_hbm.at[pl.ds(base + g * CHUNK, CHUNK)],
            osem.at[b],
        )

    for b in range(LOOK):
        gather(b, b).start()

    for b in range(NBUF):
        g = b
        gather(g, b).wait()
        out_copy(g, b).start()
        h = g + LOOK
        hb = h % NBUF
        if h >= NBUF:
            out_copy(h - NBUF, hb).wait()
        gather(h, hb).start()

    def round_body(i, carry):
        for b in range(NBUF):
            g = i * NBUF + b
            hb = (b + LOOK) % NBUF
            gather(g, b).wait()
            out_copy(g, b).start()
            out_copy(g - LOOK, hb).wait()
            gather(g + LOOK, hb).start()
        return carry

    lax.fori_loop(1, NROUND - 1, round_body, 0)

    for b in range(NBUF):
        g = (NROUND - 1) * NBUF + b
        gather(g, b).wait()
        out_copy(g, b).start()
        h = g + LOOK
        if h < NCHUNK:
            hb = h % NBUF
            out_copy(g - LOOK, hb).wait()
            gather(h, hb).start()

    for b in range(NBUF):
        g = (NROUND - 1) * NBUF + b
        out_copy(g, b).wait()


@jax.jit
def kernel(x, emb):
    idx = x.reshape(-1).astype(jnp.int32)
    mesh = plsc.VectorSubcoreMesh(core_axis_name="c", subcore_axis_name="s")
    out = pl.kernel(
        _gather_kernel,
        out_type=jax.ShapeDtypeStruct((B_TOTAL, D_EMBED), jnp.float32),
        mesh=mesh,
        scratch_types=[
            pltpu.VMEM((B_PER_W,), jnp.int32),
            pltpu.VMEM((NBUF, CHUNK, D_EMBED), jnp.float32),
            pltpu.SemaphoreType.DMA((NBUF,)),
            pltpu.SemaphoreType.DMA((NBUF,)),
        ],
        compiler_params=pltpu.CompilerParams(use_tc_tiling_on_sc=False),
    )(idx, emb)
    return out.reshape(x.shape[0], x.shape[1], D_EMBED)


# speedup vs baseline: 1.0489x; 1.0026x over previous
import jax
import jax.numpy as jnp
from jax import lax
from jax.experimental import pallas as pl
from jax.experimental.pallas import tpu as pltpu
from jax.experimental.pallas import tpu_sc as plsc

B_TOTAL = 1024 * 200
D_EMBED = 64
NW = 32
B_PER_W = B_TOTAL // NW
CHUNK = 128
NCHUNK = B_PER_W // CHUNK
NBUF = 10
LOOK = 5
NROUND = NCHUNK // NBUF


GPB = CHUNK // 16


def _gather_kernel(idx_hbm, table_hbm, out_hbm, idx_v, rows_v, gsem, osem):
    wid = lax.axis_index("s") * 2 + lax.axis_index("c")
    base = wid * B_PER_W
    pltpu.sync_copy(idx_hbm.at[pl.ds(base, B_PER_W)], idx_v)

    class _G:
        def __init__(self, g, b):
            self.g, self.b = g, b

        def start(self):
            for j in range(GPB):
                ivec = idx_v[pl.ds(self.g * CHUNK + j * 16, 16)]
                pltpu.make_async_copy(
                    table_hbm.at[ivec],
                    rows_v.at[self.b].at[pl.ds(j * 16, 16)],
                    gsem.at[self.b],
                ).start()

        def wait(self):
            pltpu.make_async_copy(
                table_hbm.at[pl.ds(0, CHUNK)],
                rows_v.at[self.b],
                gsem.at[self.b],
            ).wait()

    def gather(g, b):
        return _G(g, b)

    def out_copy(g, b):
        return pltpu.make_async_copy(
            rows_v.at[b],
            out_hbm.at[pl.ds(base + g * CHUNK, CHUNK)],
            osem.at[b],
        )

    for b in range(LOOK):
        gather(b, b).start()

    for b in range(NBUF):
        g = b
        gather(g, b).wait()
        out_copy(g, b).start()
        h = g + LOOK
        hb = h % NBUF
        if h >= NBUF:
            out_copy(h - NBUF, hb).wait()
        gather(h, hb).start()

    def round_body(i, carry):
        for b in range(NBUF):
            g = i * NBUF + b
            hb = (b + LOOK) % NBUF
            gather(g, b).wait()
            out_copy(g, b).start()
            out_copy(g - LOOK, hb).wait()
            gather(g + LOOK, hb).start()
        return carry

    lax.fori_loop(1, NROUND - 1, round_body, 0)

    for b in range(NBUF):
        g = (NROUND - 1) * NBUF + b
        gather(g, b).wait()
        out_copy(g, b).start()
        h = g + LOOK
        if h < NCHUNK:
            hb = h % NBUF
            out_copy(g - LOOK, hb).wait()
            gather(h, hb).start()

    for b in range(NBUF):
        g = (NROUND - 1) * NBUF + b
        out_copy(g, b).wait()


@jax.jit
def kernel(x, emb):
    idx = x.reshape(-1).astype(jnp.int32)
    mesh = plsc.VectorSubcoreMesh(core_axis_name="c", subcore_axis_name="s")
    out = pl.kernel(
        _gather_kernel,
        out_type=jax.ShapeDtypeStruct((B_TOTAL, D_EMBED), jnp.float32),
        mesh=mesh,
        scratch_types=[
            pltpu.VMEM((B_PER_W,), jnp.int32),
            pltpu.VMEM((NBUF, CHUNK, D_EMBED), jnp.float32),
            pltpu.SemaphoreType.DMA((NBUF,)),
            pltpu.SemaphoreType.DMA((NBUF,)),
        ],
        compiler_params=pltpu.CompilerParams(use_tc_tiling_on_sc=False),
    )(idx, emb)
    return out.reshape(x.shape[0], x.shape[1], D_EMBED)


# speedup vs baseline: 1.1100x; 1.0583x over previous
import jax
import jax.numpy as jnp
from jax import lax
from jax.experimental import pallas as pl
from jax.experimental.pallas import tpu as pltpu
from jax.experimental.pallas import tpu_sc as plsc

B_TOTAL = 1024 * 200
D_EMBED = 64
D_PAD = 128
NW = 32
B_PER_W = B_TOTAL // NW
CHUNK = 128
NCHUNK = B_PER_W // CHUNK
NBUF = 5
LOOK = 3
NROUND = NCHUNK // NBUF
GPB = CHUNK // 16


def _gather_kernel(idx_hbm, table_hbm, out_hbm, idx_v, rows_v, gsem, osem):
    wid = lax.axis_index("s") * 2 + lax.axis_index("c")
    base = wid * B_PER_W
    pltpu.sync_copy(idx_hbm.at[pl.ds(base, B_PER_W)], idx_v)

    class _G:
        def __init__(self, g, b):
            self.g, self.b = g, b

        def start(self):
            for j in range(GPB):
                ivec = idx_v[pl.ds(self.g * CHUNK + j * 16, 16)]
                pltpu.make_async_copy(
                    table_hbm.at[ivec],
                    rows_v.at[self.b].at[pl.ds(j * 16, 16)],
                    gsem.at[self.b],
                ).start()

        def wait(self):
            pltpu.make_async_copy(
                table_hbm.at[pl.ds(0, CHUNK)],
                rows_v.at[self.b],
                gsem.at[self.b],
            ).wait()

    def gather(g, b):
        return _G(g, b)

    def out_copy(g, b):
        return pltpu.make_async_copy(
            rows_v.at[b].at[:, pl.ds(0, D_EMBED)],
            out_hbm.at[pl.ds(base + g * CHUNK, CHUNK)],
            osem.at[b],
        )

    for b in range(LOOK):
        gather(b, b).start()

    for b in range(NBUF):
        g = b
        gather(g, b).wait()
        out_copy(g, b).start()
        h = g + LOOK
        hb = h % NBUF
        if h >= NBUF:
            out_copy(h - NBUF, hb).wait()
        gather(h, hb).start()

    def round_body(i, carry):
        for b in range(NBUF):
            g = i * NBUF + b
            hb = (b + LOOK) % NBUF
            gather(g, b).wait()
            out_copy(g, b).start()
            out_copy(g - (NBUF - LOOK), hb).wait()
            gather(g + LOOK, hb).start()
        return carry

    lax.fori_loop(1, NROUND - 1, round_body, 0)

    for b in range(NBUF):
        g = (NROUND - 1) * NBUF + b
        gather(g, b).wait()
        out_copy(g, b).start()
        h = g + LOOK
        if h < NCHUNK:
            hb = h % NBUF
            out_copy(g - (NBUF - LOOK), hb).wait()
            gather(h, hb).start()

    for b in range(NBUF):
        g = (NROUND - 1) * NBUF + b
        out_copy(g, b).wait()


@jax.jit
def kernel(x, emb):
    idx = x.reshape(-1).astype(jnp.int32)
    embp = jnp.pad(emb, ((0, 0), (0, D_PAD - D_EMBED)))
    mesh = plsc.VectorSubcoreMesh(core_axis_name="c", subcore_axis_name="s")
    out = pl.kernel(
        _gather_kernel,
        out_type=jax.ShapeDtypeStruct((B_TOTAL, D_EMBED), jnp.float32),
        mesh=mesh,
        scratch_types=[
            pltpu.VMEM((B_PER_W,), jnp.int32),
            pltpu.VMEM((NBUF, CHUNK, D_PAD), jnp.float32),
            pltpu.SemaphoreType.DMA((NBUF,)),
            pltpu.SemaphoreType.DMA((NBUF,)),
        ],
        compiler_params=pltpu.CompilerParams(use_tc_tiling_on_sc=False),
    )(idx, embp)
    return out.reshape(x.shape[0], x.shape[1], D_EMBED)
